# baseline (speedup 1.0000x reference)
import jax
import jax.numpy as jnp
from jax.experimental import pallas as pl
from jax.experimental.pallas import tpu as pltpu

_C = 2000
_B = 4


def _copy_body(x_ref, e_ref, xo_ref, eo_ref, vbuf, in_sems, out_sems):
    e2 = e_ref
    eo2 = eo_ref

    items = []
    for i in range(10000 // _C):
        s = pl.ds(jnp.int32(i * _C), _C)
        items.append((x_ref.at[s, :], xo_ref.at[s, :]))
    for i in range(40000 // _C):
        s = pl.ds(jnp.int32(i * _C), _C)
        items.append((e2.at[s, :], eo2.at[s, :]))

    out_copies = {}
    for i, (src, dst) in enumerate(items):
        b = jnp.int32(i % _B)
        if i >= _B:
            out_copies[i - _B].wait()
        c_in = pltpu.make_async_copy(src, vbuf.at[b], in_sems.at[b])
        c_in.start()
        c_in.wait()
        c_out = pltpu.make_async_copy(vbuf.at[b], dst, out_sems.at[b])
        c_out.start()
        out_copies[i] = c_out
    for i in range(len(items) - _B, len(items)):
        out_copies[i].wait()


def kernel(x, edge_index, edge_attr):
    del edge_index
    e2 = edge_attr.reshape(40000, 128)
    x_out, e_out = pl.pallas_call(
        _copy_body,
        in_specs=[
            pl.BlockSpec(memory_space=pl.ANY),
            pl.BlockSpec(memory_space=pl.ANY),
        ],
        out_specs=[
            pl.BlockSpec(memory_space=pl.ANY),
            pl.BlockSpec(memory_space=pl.ANY),
        ],
        out_shape=[
            jax.ShapeDtypeStruct(x.shape, x.dtype),
            jax.ShapeDtypeStruct((40000, 128), edge_attr.dtype),
        ],
        scratch_shapes=[
            pltpu.VMEM((_B, _C, 128), jnp.float32),
            pltpu.SemaphoreType.DMA((_B,)),
            pltpu.SemaphoreType.DMA((_B,)),
        ],
    )(x, e2)
    return (x_out, e_out.reshape(320000, 16))


# speedup vs baseline: 1.0750x; 1.0750x over previous
import functools

import jax
import jax.numpy as jnp
from jax import lax
from jax.experimental import pallas as pl
from jax.experimental.pallas import tpu as pltpu
from jax.experimental.pallas import tpu_sc as plsc

_N_WORKERS = 32
_E_ROWS = 320000
_ROWS_PER_W = _E_ROWS // _N_WORKERS
_ECHUNK = 2000
_NCHUNK = _ROWS_PER_W // _ECHUNK


@functools.partial(
    pl.kernel,
    out_type=jax.ShapeDtypeStruct((_E_ROWS, 16), jnp.float32),
    mesh=plsc.VectorSubcoreMesh(core_axis_name="c", subcore_axis_name="s"),
    compiler_params=pltpu.CompilerParams(use_tc_tiling_on_sc=False),
    scratch_types=[
        pltpu.VMEM((2, _ECHUNK, 16), jnp.float32),
        pltpu.SemaphoreType.DMA,
        pltpu.SemaphoreType.DMA,
        pltpu.SemaphoreType.DMA,
        pltpu.SemaphoreType.DMA,
    ],
)
def _sc_copy_edge(e_hbm, out_hbm, buf, in0, in1, out0, out1):
    wid = lax.axis_index("s") * 2 + lax.axis_index("c")
    base = pl.multiple_of(wid * _ROWS_PER_W, 8)
    in_sems = (in0, in1)
    out_sems = (out0, out1)
    out_copies = {}
    for j in range(_NCHUNK):
        b = j % 2
        row = pl.multiple_of(base + jnp.int32(j * _ECHUNK), 8)
        if j >= 2:
            out_copies[j - 2].wait()
        c_in = pltpu.make_async_copy(
            e_hbm.at[pl.ds(row, _ECHUNK), :], buf.at[jnp.int32(b)], in_sems[b])
        c_in.start()
        c_in.wait()
        c_out = pltpu.make_async_copy(
            buf.at[jnp.int32(b)], out_hbm.at[pl.ds(row, _ECHUNK), :],
            out_sems[b])
        c_out.start()
        out_copies[j] = c_out
    out_copies[_NCHUNK - 2].wait()
    out_copies[_NCHUNK - 1].wait()


_XGRID = 10
_XB = 10000 // _XGRID


def _tc_copy_body(x_ref, xo_ref):
    xo_ref[...] = x_ref[...]


def _tc_copy_x(x):
    return pl.pallas_call(
        _tc_copy_body,
        grid=(_XGRID,),
        in_specs=[pl.BlockSpec((_XB, 128), lambda i: (i, jnp.int32(0)))],
        out_specs=pl.BlockSpec((_XB, 128), lambda i: (i, jnp.int32(0))),
        out_shape=jax.ShapeDtypeStruct((10000, 128), jnp.float32),
        compiler_params=pltpu.CompilerParams(
            dimension_semantics=("arbitrary",),
        ),
    )(x)


def kernel(x, edge_index, edge_attr):
    del edge_index
    e_out = _sc_copy_edge(edge_attr)
    x_out = _tc_copy_x(x)
    return (x_out, e_out)


# speedup vs baseline: 1.0752x; 1.0001x over previous
import functools

import jax
import jax.numpy as jnp
from jax import lax
from jax.experimental import pallas as pl
from jax.experimental.pallas import tpu as pltpu
from jax.experimental.pallas import tpu_sc as plsc

_N_WORKERS = 32
_E_ELEMS = 320000 * 16
_PER_W = _E_ELEMS // _N_WORKERS
_ECHUNK = 32000
_NCHUNK = _PER_W // _ECHUNK


@functools.partial(
    pl.kernel,
    out_type=jax.ShapeDtypeStruct((_E_ELEMS,), jnp.float32),
    mesh=plsc.VectorSubcoreMesh(core_axis_name="c", subcore_axis_name="s"),
    compiler_params=pltpu.CompilerParams(use_tc_tiling_on_sc=False),
    scratch_types=[
        pltpu.VMEM((2, _ECHUNK), jnp.float32),
        pltpu.SemaphoreType.DMA,
        pltpu.SemaphoreType.DMA,
        pltpu.SemaphoreType.DMA,
        pltpu.SemaphoreType.DMA,
    ],
)
def _sc_copy_edge(e_hbm, out_hbm, buf, in0, in1, out0, out1):
    wid = lax.axis_index("s") * 2 + lax.axis_index("c")
    base = pl.multiple_of(wid * _PER_W, 8)
    in_sems = (in0, in1)
    out_sems = (out0, out1)
    out_copies = {}
    for j in range(_NCHUNK):
        b = j % 2
        off = pl.multiple_of(base + jnp.int32(j * _ECHUNK), 8)
        if j >= 2:
            out_copies[j - 2].wait()
        c_in = pltpu.make_async_copy(
            e_hbm.at[pl.ds(off, _ECHUNK)], buf.at[jnp.int32(b)], in_sems[b])
        c_in.start()
        c_in.wait()
        c_out = pltpu.make_async_copy(
            buf.at[jnp.int32(b)], out_hbm.at[pl.ds(off, _ECHUNK)],
            out_sems[b])
        c_out.start()
        out_copies[j] = c_out
    out_copies[_NCHUNK - 2].wait()
    out_copies[_NCHUNK - 1].wait()


_XGRID = 10
_XB = 10000 // _XGRID


def _tc_copy_body(x_ref, xo_ref):
    xo_ref[...] = x_ref[...]


def _tc_copy_x(x):
    return pl.pallas_call(
        _tc_copy_body,
        grid=(_XGRID,),
        in_specs=[pl.BlockSpec((_XB, 128), lambda i: (i, jnp.int32(0)))],
        out_specs=pl.BlockSpec((_XB, 128), lambda i: (i, jnp.int32(0))),
        out_shape=jax.ShapeDtypeStruct((10000, 128), jnp.float32),
        compiler_params=pltpu.CompilerParams(
            dimension_semantics=("arbitrary",),
        ),
    )(x)


def kernel(x, edge_index, edge_attr):
    del edge_index
    e_out = _sc_copy_edge(edge_attr.reshape(_E_ELEMS))
    x_out = _tc_copy_x(x)
    return (x_out, e_out.reshape(320000, 16))


# speedup vs baseline: 1.1375x; 1.0580x over previous
import jax
import jax.numpy as jnp
from jax.experimental import pallas as pl
from jax.experimental.pallas import tpu as pltpu

_EK = 8
_EB = 4000
_ECHUNKS = 320000 // _EB
_XB = 1000
_XCHUNKS = 10000 // _XB
_DEPTH = 2


def _copy_body(x_ref, e_ref, xo_ref, eo_ref, xbuf, ebuf, xsems, esems_i,
               esems_o, xsem_o):
    def chunk(s, r):
        if s == 0:
            if r >= _XCHUNKS:
                return None
            off = jnp.int32(r * _XB)
            return (x_ref.at[pl.ds(off, _XB), :],
                    xo_ref.at[pl.ds(off, _XB), :],
                    xbuf.at[jnp.int32(r % _DEPTH)],
                    xsems.at[jnp.int32(r % _DEPTH)],
                    xsem_o.at[jnp.int32(r % _DEPTH)])
        k = s - 1
        c = r * _EK + k
        if c >= _ECHUNKS:
            return None
        off = jnp.int32(c * _EB)
        return (e_ref.at[pl.ds(off, _EB), :],
                eo_ref.at[pl.ds(off, _EB), :],
                ebuf.at[jnp.int32(k), jnp.int32(r % _DEPTH)],
                esems_i.at[jnp.int32(k), jnp.int32(r % _DEPTH)],
                esems_o.at[jnp.int32(k), jnp.int32(r % _DEPTH)])

    n_streams = 1 + _EK
    rounds = max(_XCHUNKS, -(-_ECHUNKS // _EK))
    in_copies = {}
    out_copies = {}

    for r in range(_DEPTH):
        for s in range(n_streams):
            it = chunk(s, r)
            if it is None:
                continue
            src, dst, buf, sem_i, sem_o = it
            c = pltpu.make_async_copy(src, buf, sem_i)
            c.start()
            in_copies[(s, r)] = c

    for r in range(rounds):
        for s in range(n_streams):
            it = chunk(s, r)
            if it is None:
                continue
            src, dst, buf, sem_i, sem_o = it
            in_copies.pop((s, r)).wait()
            co = pltpu.make_async_copy(buf, dst, sem_o)
            co.start()
            out_copies[(s, r)] = co
            nxt = r + _DEPTH
            it2 = chunk(s, nxt)
            if it2 is not None:
                out_copies.pop((s, r)).wait()
                src2, _, buf2, sem_i2, _ = it2
                ci = pltpu.make_async_copy(src2, buf2, sem_i2)
                ci.start()
                in_copies[(s, nxt)] = ci

    for key in list(out_copies):
        out_copies.pop(key).wait()


def kernel(x, edge_index, edge_attr):
    del edge_index
    x_out, e_out = pl.pallas_call(
        _copy_body,
        in_specs=[
            pl.BlockSpec(memory_space=pl.ANY),
            pl.BlockSpec(memory_space=pl.ANY),
        ],
        out_specs=[
            pl.BlockSpec(memory_space=pl.ANY),
            pl.BlockSpec(memory_space=pl.ANY),
        ],
        out_shape=[
            jax.ShapeDtypeStruct(x.shape, x.dtype),
            jax.ShapeDtypeStruct(edge_attr.shape, edge_attr.dtype),
        ],
        scratch_shapes=[
            pltpu.VMEM((_DEPTH, _XB, 128), jnp.float32),
            pltpu.VMEM((_EK, _DEPTH, _EB, 16), jnp.float32),
            pltpu.SemaphoreType.DMA((_DEPTH,)),
            pltpu.SemaphoreType.DMA((_EK, _DEPTH)),
            pltpu.SemaphoreType.DMA((_EK, _DEPTH)),
            pltpu.SemaphoreType.DMA((_DEPTH,)),
        ],
    )(x, edge_attr)
    return (x_out, e_out)
